# Initial kernel scaffold; baseline (speedup 1.0000x reference)
import jax
import jax.numpy as jnp
from jax.experimental import pallas as pl


def kernel(h, edge_indexT, D):
    raise NotImplementedError("write your pallas kernel here")


# speedup vs baseline: 5.8674x; 5.8674x over previous
import functools

import jax
import jax.numpy as jnp
from jax import lax
from jax.experimental import pallas as pl
from jax.experimental.pallas import tpu as pltpu
from jax.experimental.pallas import tpu_sc as plsc

N = 10000
NP = 10240
E = 320000
FH = 64
GAMMA = 0.1
EPT = 20000
C = 128
GSZ = 8
NCH = 160
EPTP = NCH * C
RPT = 640
RC = 128
PAD_ROW = N + 16

_mesh = plsc.VectorSubcoreMesh(core_axis_name="c", subcore_axis_name="s")


@functools.partial(
    pl.kernel,
    mesh=_mesh,
    out_type=jax.ShapeDtypeStruct((2, N, FH), jnp.float32),
    scratch_types=[
        pltpu.VMEM_SHARED((NP, FH), jnp.float32),
        pltpu.VMEM_SHARED((NP, FH), jnp.float32),
        pltpu.VMEM((GSZ, C), jnp.int32),
        pltpu.VMEM((GSZ, C), jnp.int32),
        pltpu.VMEM((RC, FH), jnp.float32),
        pltpu.VMEM((RC, FH), jnp.float32),
        pltpu.VMEM((RPT + 16,), jnp.float32),
    ],
)
def _smooth(hT, srcp, dstp, d2, out, h_sp, agg_sp, srcC, dstC,
            gbufA, gbufB, f2buf):
    cid = lax.axis_index("c")
    sid = lax.axis_index("s")
    r0 = sid * RPT

    @pl.when(sid < 15)
    def _():
        pltpu.sync_copy(hT.at[cid].at[pl.ds(r0, RPT)], h_sp.at[pl.ds(r0, RPT)])

    @pl.when(sid == 15)
    def _():
        pltpu.sync_copy(hT.at[cid].at[pl.ds(r0, N - 15 * RPT)],
                        h_sp.at[pl.ds(r0, N - 15 * RPT)])

    pltpu.sync_copy(d2.at[sid], f2buf.at[pl.ds(0, RPT)])

    def _f2(i, _):
        sl = pl.ds(i * 16, 16)
        f2buf[sl] = 1.0 / (1.0 + GAMMA * f2buf[sl])
        return _
    lax.fori_loop(0, RPT // 16, _f2, None)

    for _t in range(2):
        def _zb(r, _):
            for q in range(FH // 16):
                gbufB[r, pl.ds(q * 16, 16)] = jnp.zeros((16,), jnp.float32)
            return _
        lax.fori_loop(0, RC, _zb, None)
        for k in range(RPT // RC):
            pltpu.sync_copy(gbufB, agg_sp.at[pl.ds(r0 + k * RC, RC)])
        plsc.subcore_barrier()

        def _edge(g, _):
            pltpu.sync_copy(srcp.at[sid].at[pl.ds(g * GSZ, GSZ)], srcC)
            pltpu.sync_copy(dstp.at[sid].at[pl.ds(g * GSZ, GSZ)], dstC)
            for jj in range(GSZ):
                pltpu.sync_copy(h_sp.at[srcC.at[jj]], gbufA)
                pltpu.sync_copy(gbufA, agg_sp.at[dstC.at[jj]], add=True)
            return _
        lax.fori_loop(0, NCH // GSZ, _edge, None)
        plsc.subcore_barrier()

        for k in range(RPT // RC):
            rk = r0 + k * RC
            pltpu.sync_copy(h_sp.at[pl.ds(rk, RC)], gbufA)
            pltpu.sync_copy(agg_sp.at[pl.ds(rk, RC)], gbufB)

            def _row(r, _, k=k):
                sv = f2buf[pl.ds(k * RC + r, 16)]
                s = jnp.full((16,), sv[0], jnp.float32)
                for q in range(FH // 16):
                    sl = pl.ds(q * 16, 16)
                    gbufA[r, sl] = (gbufA[r, sl] + GAMMA * gbufB[r, sl]) * s
                return _
            lax.fori_loop(0, RC, _row, None)
            pltpu.sync_copy(gbufA, h_sp.at[pl.ds(rk, RC)])
        plsc.subcore_barrier()

    @pl.when(sid < 15)
    def _():
        pltpu.sync_copy(h_sp.at[pl.ds(r0, RPT)], out.at[cid].at[pl.ds(r0, RPT)])

    @pl.when(sid == 15)
    def _():
        pltpu.sync_copy(h_sp.at[pl.ds(r0, N - 15 * RPT)],
                        out.at[cid].at[pl.ds(r0, N - 15 * RPT)])


def kernel(h, edge_indexT, D):
    src = edge_indexT[0].reshape(16, EPT)
    dst = edge_indexT[1].reshape(16, EPT)
    srcp = jnp.pad(src, ((0, 0), (0, EPTP - EPT))).reshape(16, NCH, C)
    dstp = jnp.pad(dst, ((0, 0), (0, EPTP - EPT)),
                   constant_values=PAD_ROW).reshape(16, NCH, C)
    d2 = jnp.pad(D, (0, NP - N)).reshape(16, RPT)
    hT = jnp.transpose(h.reshape(N, 2, FH), (1, 0, 2))
    outT = _smooth(hT, srcp, dstp, d2)
    return jnp.transpose(outT, (1, 0, 2)).reshape(N, 2 * FH)


# speedup vs baseline: 6.0380x; 1.0291x over previous
import functools

import jax
import jax.numpy as jnp
from jax import lax
from jax.experimental import pallas as pl
from jax.experimental.pallas import tpu as pltpu
from jax.experimental.pallas import tpu_sc as plsc

N = 10000
NP = 10240
E = 320000
FH = 64
GAMMA = 0.1
EPT = 20000
C = 128
GSZ = 16
NCH = 160
EPTP = NCH * C
RPT = 640
RC = 128
PAD_ROW = N + 16

_mesh = plsc.VectorSubcoreMesh(core_axis_name="c", subcore_axis_name="s")


@functools.partial(
    pl.kernel,
    mesh=_mesh,
    out_type=jax.ShapeDtypeStruct((2, N, FH), jnp.float32),
    scratch_types=[
        pltpu.VMEM_SHARED((NP, FH), jnp.float32),
        pltpu.VMEM_SHARED((NP, FH), jnp.float32),
        pltpu.VMEM((GSZ, C), jnp.int32),
        pltpu.VMEM((GSZ, C), jnp.int32),
        pltpu.VMEM((RC, FH), jnp.float32),
        pltpu.VMEM((RC, FH), jnp.float32),
        pltpu.VMEM((RPT + 16,), jnp.float32),
    ],
)
def _smooth(hT, srcp, dstp, d2, out, h_sp, agg_sp, srcC, dstC,
            gbufA, gbufB, f2buf):
    cid = lax.axis_index("c")
    sid = lax.axis_index("s")
    r0 = sid * RPT

    @pl.when(sid < 15)
    def _():
        pltpu.sync_copy(hT.at[cid].at[pl.ds(r0, RPT)], h_sp.at[pl.ds(r0, RPT)])

    @pl.when(sid == 15)
    def _():
        pltpu.sync_copy(hT.at[cid].at[pl.ds(r0, N - 15 * RPT)],
                        h_sp.at[pl.ds(r0, N - 15 * RPT)])

    pltpu.sync_copy(d2.at[sid], f2buf.at[pl.ds(0, RPT)])

    def _f2(i, _):
        sl = pl.ds(i * 16, 16)
        f2buf[sl] = 1.0 / (1.0 + GAMMA * f2buf[sl])
        return _
    lax.fori_loop(0, RPT // 16, _f2, None)

    for _t in range(2):
        def _zb(r, _):
            for q in range(FH // 16):
                gbufB[r, pl.ds(q * 16, 16)] = jnp.zeros((16,), jnp.float32)
            return _
        lax.fori_loop(0, RC, _zb, None)
        for k in range(RPT // RC):
            pltpu.sync_copy(gbufB, agg_sp.at[pl.ds(r0 + k * RC, RC)])
        plsc.subcore_barrier()

        def _edge(g, _):
            pltpu.sync_copy(srcp.at[sid].at[pl.ds(g * GSZ, GSZ)], srcC)
            pltpu.sync_copy(dstp.at[sid].at[pl.ds(g * GSZ, GSZ)], dstC)
            for jj in range(GSZ):
                buf = gbufA if jj % 2 == 0 else gbufB
                pltpu.sync_copy(h_sp.at[srcC.at[jj]], buf)
                pltpu.sync_copy(buf, agg_sp.at[dstC.at[jj]], add=True)
            return _
        lax.fori_loop(0, NCH // GSZ, _edge, None)
        plsc.subcore_barrier()

        for k in range(RPT // RC):
            rk = r0 + k * RC
            pltpu.sync_copy(h_sp.at[pl.ds(rk, RC)], gbufA)
            pltpu.sync_copy(agg_sp.at[pl.ds(rk, RC)], gbufB)

            def _row(r, _, k=k):
                sv = f2buf[pl.ds(k * RC + r, 16)]
                s = jnp.full((16,), sv[0], jnp.float32)
                for q in range(FH // 16):
                    sl = pl.ds(q * 16, 16)
                    gbufA[r, sl] = (gbufA[r, sl] + GAMMA * gbufB[r, sl]) * s
                return _
            lax.fori_loop(0, RC, _row, None)
            pltpu.sync_copy(gbufA, h_sp.at[pl.ds(rk, RC)])
        plsc.subcore_barrier()

    @pl.when(sid < 15)
    def _():
        pltpu.sync_copy(h_sp.at[pl.ds(r0, RPT)], out.at[cid].at[pl.ds(r0, RPT)])

    @pl.when(sid == 15)
    def _():
        pltpu.sync_copy(h_sp.at[pl.ds(r0, N - 15 * RPT)],
                        out.at[cid].at[pl.ds(r0, N - 15 * RPT)])


def kernel(h, edge_indexT, D):
    src = edge_indexT[0].reshape(16, EPT)
    dst = edge_indexT[1].reshape(16, EPT)
    srcp = jnp.pad(src, ((0, 0), (0, EPTP - EPT))).reshape(16, NCH, C)
    dstp = jnp.pad(dst, ((0, 0), (0, EPTP - EPT)),
                   constant_values=PAD_ROW).reshape(16, NCH, C)
    d2 = jnp.pad(D, (0, NP - N)).reshape(16, RPT)
    hT = jnp.transpose(h.reshape(N, 2, FH), (1, 0, 2))
    outT = _smooth(hT, srcp, dstp, d2)
    return jnp.transpose(outT, (1, 0, 2)).reshape(N, 2 * FH)
